# baseline (speedup 1.0000x reference)
import jax
import jax.numpy as jnp
from jax.experimental import pallas as pl
from jax.experimental.pallas import tpu as pltpu

GAMMA = 2.0
BLOCK_R = 512


def kernel(inputs, targets, alpha):
    B, Q, N = inputs.shape
    R = B * Q
    x = inputs.reshape(R, N)
    t3 = targets.reshape(R // BLOCK_R, 1, BLOCK_R)
    a2 = alpha.reshape(1, N)

    def body(x_ref, t_ref, a_ref, ids_ref, out_ref):
        i = pl.program_id(0)
        xb = x_ref[...]
        t = t_ref[0, 0, :]
        s = jnp.sum(jnp.exp(xb), axis=1, keepdims=True)
        mask = ids_ref[...] == t[:, None]
        xt = jnp.sum(jnp.where(mask, xb, 0.0), axis=1, keepdims=True)
        at = jnp.sum(jnp.where(mask, a_ref[...], 0.0), axis=1, keepdims=True)
        logp = xt - jnp.log(s)
        p = jnp.exp(logp)
        q1 = 1.0 - p
        part = jnp.sum(-at * q1 * q1 * logp) * (1.0 / R)

        @pl.when(i == 0)
        def _():
            out_ref[0, 0] = 0.0

        out_ref[0, 0] += part

    out = pl.pallas_call(
        body,
        grid=(R // BLOCK_R,),
        in_specs=[
            pl.BlockSpec((BLOCK_R, N), lambda i: (i, 0)),
            pl.BlockSpec((1, 1, BLOCK_R), lambda i: (i, 0, 0)),
            pl.BlockSpec((1, N), lambda i: (0, 0)),
            pl.BlockSpec((1, N), lambda i: (0, 0)),
        ],
        out_specs=pl.BlockSpec(memory_space=pltpu.SMEM),
        out_shape=jax.ShapeDtypeStruct((1, 1), jnp.float32),
    )(x, t3, a2, jnp.arange(N, dtype=jnp.int32).reshape(1, N))
    return out[0, 0]
